# baseline (speedup 1.0000x reference)
import functools

import jax
import jax.numpy as jnp
from jax import lax
from jax.experimental import pallas as pl
from jax.experimental.pallas import tpu as pltpu
from jax.experimental.pallas import tpu_sc as plsc

_CHUNKS = 8
_BLK = 512


@functools.partial(jax.jit, static_argnums=(2, 3, 4, 5, 6))
def _sc_gather(x2, nbr1, co, CN, D, K, G):
    NC, NS = 2, 16
    NPW = CN // (NC * NS)
    GK = G * K
    n_groups = NPW // G
    NI, NG = 4, 4
    e_chunk = co * CN * K

    mesh = plsc.VectorSubcoreMesh(core_axis_name="c", subcore_axis_name="s")

    @functools.partial(
        pl.kernel,
        mesh=mesh,
        out_type=jax.ShapeDtypeStruct((CN * K, D), jnp.float32),
        scratch_types=[
            pltpu.VMEM((NI, GK), jnp.int32),
            pltpu.VMEM((NG, GK, D), jnp.float32),
        ] + [pltpu.SemaphoreType.DMA] * (NI + 2 * NG),
    )
    def k(x_hbm, nbr_hbm, out_hbm, idx_v, gat_v, *sems):
        isem = sems[:NI]
        gsem = sems[NI:NI + NG]
        ssem = sems[NI + NG:]
        wid = lax.axis_index("s") * NC + lax.axis_index("c")
        e0w = wid * NPW * K

        def issue_idx(g, si):
            eg = e_chunk + e0w + g * GK
            pltpu.async_copy(nbr_hbm.at[pl.ds(eg, GK)], idx_v.at[si], isem[si])

        def wait_idx(si):
            pltpu.make_async_copy(
                nbr_hbm.at[pl.ds(0, GK)], idx_v.at[si], isem[si]).wait()

        def issue_gather(g, sg, si):
            pltpu.async_copy(x_hbm.at[idx_v.at[si]], gat_v.at[sg], gsem[sg])

        def wait_gather(sg, si):
            pltpu.make_async_copy(
                x_hbm.at[idx_v.at[si]], gat_v.at[sg], gsem[sg]).wait()

        def issue_out(g, sg):
            eg = e0w + g * GK
            pltpu.async_copy(gat_v.at[sg], out_hbm.at[pl.ds(eg, GK)], ssem[sg])

        def wait_out(sg):
            pltpu.make_async_copy(
                gat_v.at[sg], out_hbm.at[pl.ds(0, GK)], ssem[sg]).wait()

        issue_idx(0, 0)
        issue_idx(1, 1)
        wait_idx(0)
        issue_gather(0, 0, 0)

        def quad_body(gg, car):
            for u in range(NG):
                g2 = gg * NG + u

                @pl.when(g2 + 2 < n_groups)
                def _():
                    issue_idx(g2 + 2, (u + 2) % NI)

                @pl.when(g2 + 1 < n_groups)
                def _():
                    wait_idx((u + 1) % NI)

                    @pl.when(g2 + 1 >= NG)
                    def _():
                        wait_out((u + 1) % NG)

                    issue_gather(g2 + 1, (u + 1) % NG, (u + 1) % NI)

                wait_gather(u, u % NI)
                issue_out(g2, u)
            return car

        lax.fori_loop(0, n_groups // NG, quad_body, 0)
        for t in range(min(NG, n_groups)):
            wait_out((n_groups - 1 - t) % NG)

    return k(x2, nbr1)


def _tc_body(x_ref, nbr_ref, alias_ref, out_ref):
    del alias_ref
    xb = x_ref[...]
    K, D = nbr_ref.shape[1], x_ref.shape[1]
    br = lax.broadcast_in_dim(xb, (xb.shape[0], K, D), (0, 2))
    out_ref[:, :, 0:D] = br
    out_ref[:, :, D:2 * D] = nbr_ref[...] - br


def _tc_body0(x_ref, nbr_ref, out_ref):
    _tc_body(x_ref, nbr_ref, None, out_ref)


@functools.partial(jax.jit, static_argnums=(3, 4, 5, 6, 7))
def _tc_stage(x2, nbr_c, prev_out, co, BN, D, K, CN):
    nblk = CN // _BLK
    base = co * nblk
    grid = (nblk,)
    x_spec = pl.BlockSpec((_BLK, D), lambda i, _b=base: (i + _b, 0))
    nbr_spec = pl.BlockSpec((_BLK, K, D), lambda i: (i, 0, 0))
    out_spec = pl.BlockSpec((_BLK, K, 2 * D), lambda i, _b=base: (i + _b, 0, 0))
    out_shape = jax.ShapeDtypeStruct((BN, K, 2 * D), jnp.float32)
    nbr3 = nbr_c.reshape(CN, K, D)
    if prev_out is None:
        return pl.pallas_call(
            _tc_body0, grid=grid,
            in_specs=[x_spec, nbr_spec],
            out_specs=out_spec, out_shape=out_shape,
        )(x2, nbr3)
    return pl.pallas_call(
        _tc_body, grid=grid,
        in_specs=[x_spec, nbr_spec, pl.BlockSpec(memory_space=pl.ANY)],
        out_specs=out_spec, out_shape=out_shape,
        input_output_aliases={2: 0},
    )(x2, nbr3, prev_out)


def kernel(X_inputs, nn_idx):
    B, N, D = X_inputs.shape
    K = nn_idx.shape[-1]
    BN = B * N
    CN = BN // _CHUNKS
    x2 = X_inputs.reshape(BN, D)
    offs = (jnp.arange(B, dtype=jnp.int32) * N).reshape(B, 1, 1)
    nbr1 = (nn_idx.astype(jnp.int32) + offs).reshape(BN * K)
    gathered = [_sc_gather(x2, nbr1, c, CN, D, K, 8) for c in range(_CHUNKS)]
    out = None
    for c in range(_CHUNKS):
        out = _tc_stage(x2, gathered[c], out, c, BN, D, K, CN)
    return out.reshape(B, N, K, 2 * D)


# speedup vs baseline: 1.0326x; 1.0326x over previous
import functools

import jax
import jax.numpy as jnp
from jax import lax
from jax.experimental import pallas as pl
from jax.experimental.pallas import tpu as pltpu
from jax.experimental.pallas import tpu_sc as plsc

_CHUNKS = 2
_BLK = 512


@functools.partial(jax.jit, static_argnums=(2, 3, 4, 5, 6))
def _sc_gather(x2, nbr1, co, CN, D, K, G):
    NC, NS = 2, 16
    NPW = CN // (NC * NS)
    GK = G * K
    n_groups = NPW // G
    NI, NG = 4, 4
    e_chunk = co * CN * K

    mesh = plsc.VectorSubcoreMesh(core_axis_name="c", subcore_axis_name="s")

    @functools.partial(
        pl.kernel,
        mesh=mesh,
        out_type=jax.ShapeDtypeStruct((CN * K, D), jnp.float32),
        scratch_types=[
            pltpu.VMEM((NI, GK), jnp.int32),
            pltpu.VMEM((NG, GK, D), jnp.float32),
        ] + [pltpu.SemaphoreType.DMA] * (NI + 2 * NG),
    )
    def k(x_hbm, nbr_hbm, out_hbm, idx_v, gat_v, *sems):
        isem = sems[:NI]
        gsem = sems[NI:NI + NG]
        ssem = sems[NI + NG:]
        wid = lax.axis_index("s") * NC + lax.axis_index("c")
        e0w = wid * NPW * K

        def issue_idx(g, si):
            eg = e_chunk + e0w + g * GK
            pltpu.async_copy(nbr_hbm.at[pl.ds(eg, GK)], idx_v.at[si], isem[si])

        def wait_idx(si):
            pltpu.make_async_copy(
                nbr_hbm.at[pl.ds(0, GK)], idx_v.at[si], isem[si]).wait()

        def issue_gather(g, sg, si):
            pltpu.async_copy(x_hbm.at[idx_v.at[si]], gat_v.at[sg], gsem[sg])

        def wait_gather(sg, si):
            pltpu.make_async_copy(
                x_hbm.at[idx_v.at[si]], gat_v.at[sg], gsem[sg]).wait()

        def issue_out(g, sg):
            eg = e0w + g * GK
            pltpu.async_copy(gat_v.at[sg], out_hbm.at[pl.ds(eg, GK)], ssem[sg])

        def wait_out(sg):
            pltpu.make_async_copy(
                gat_v.at[sg], out_hbm.at[pl.ds(0, GK)], ssem[sg]).wait()

        issue_idx(0, 0)
        issue_idx(1, 1)
        wait_idx(0)
        issue_gather(0, 0, 0)

        def quad_body(gg, car):
            for u in range(NG):
                g2 = gg * NG + u

                @pl.when(g2 + 2 < n_groups)
                def _():
                    issue_idx(g2 + 2, (u + 2) % NI)

                @pl.when(g2 + 1 < n_groups)
                def _():
                    wait_idx((u + 1) % NI)

                    @pl.when(g2 + 1 >= NG)
                    def _():
                        wait_out((u + 1) % NG)

                    issue_gather(g2 + 1, (u + 1) % NG, (u + 1) % NI)

                wait_gather(u, u % NI)
                issue_out(g2, u)
            return car

        lax.fori_loop(0, n_groups // NG, quad_body, 0)
        for t in range(min(NG, n_groups)):
            wait_out((n_groups - 1 - t) % NG)

    return k(x2, nbr1)


def _tc_body(x_ref, nbr_ref, alias_ref, out_ref):
    del alias_ref
    xb = x_ref[...]
    K, D = nbr_ref.shape[1], x_ref.shape[1]
    br = lax.broadcast_in_dim(xb, (xb.shape[0], K, D), (0, 2))
    out_ref[:, :, 0:D] = br
    out_ref[:, :, D:2 * D] = nbr_ref[...] - br


def _tc_body0(x_ref, nbr_ref, out_ref):
    _tc_body(x_ref, nbr_ref, None, out_ref)


@functools.partial(jax.jit, static_argnums=(3, 4, 5, 6, 7))
def _tc_stage(x2, nbr_c, prev_out, co, BN, D, K, CN):
    nblk = CN // _BLK
    base = co * nblk
    grid = (nblk,)
    x_spec = pl.BlockSpec((_BLK, D), lambda i, _b=base: (i + _b, 0))
    nbr_spec = pl.BlockSpec((_BLK, K, D), lambda i: (i, 0, 0))
    out_spec = pl.BlockSpec((_BLK, K, 2 * D), lambda i, _b=base: (i + _b, 0, 0))
    out_shape = jax.ShapeDtypeStruct((BN, K, 2 * D), jnp.float32)
    nbr3 = nbr_c.reshape(CN, K, D)
    if prev_out is None:
        return pl.pallas_call(
            _tc_body0, grid=grid,
            in_specs=[x_spec, nbr_spec],
            out_specs=out_spec, out_shape=out_shape,
        )(x2, nbr3)
    return pl.pallas_call(
        _tc_body, grid=grid,
        in_specs=[x_spec, nbr_spec, pl.BlockSpec(memory_space=pl.ANY)],
        out_specs=out_spec, out_shape=out_shape,
        input_output_aliases={2: 0},
    )(x2, nbr3, prev_out)


def kernel(X_inputs, nn_idx):
    B, N, D = X_inputs.shape
    K = nn_idx.shape[-1]
    BN = B * N
    CN = BN // _CHUNKS
    x2 = X_inputs.reshape(BN, D)
    offs = (jnp.arange(B, dtype=jnp.int32) * N).reshape(B, 1, 1)
    nbr1 = (nn_idx.astype(jnp.int32) + offs).reshape(BN * K)
    gathered = [_sc_gather(x2, nbr1, c, CN, D, K, 8) for c in range(_CHUNKS)]
    out = None
    for c in range(_CHUNKS):
        out = _tc_stage(x2, gathered[c], out, c, BN, D, K, CN)
    return out.reshape(B, N, K, 2 * D)


# speedup vs baseline: 1.0415x; 1.0085x over previous
import functools

import jax
import jax.numpy as jnp
from jax import lax
from jax.experimental import pallas as pl
from jax.experimental.pallas import tpu as pltpu
from jax.experimental.pallas import tpu_sc as plsc

_CHUNKS = 2
_BLK = 1024


@functools.partial(jax.jit, static_argnums=(2, 3, 4, 5, 6))
def _sc_gather(x2, nbr1, co, CN, D, K, G):
    NC, NS = 2, 16
    NPW = CN // (NC * NS)
    GK = G * K
    n_groups = NPW // G
    NI, NG = 4, 4
    e_chunk = co * CN * K

    mesh = plsc.VectorSubcoreMesh(core_axis_name="c", subcore_axis_name="s")

    @functools.partial(
        pl.kernel,
        mesh=mesh,
        out_type=jax.ShapeDtypeStruct((CN * K, D), jnp.float32),
        scratch_types=[
            pltpu.VMEM((NI, GK), jnp.int32),
            pltpu.VMEM((NG, GK, D), jnp.float32),
        ] + [pltpu.SemaphoreType.DMA] * (NI + 2 * NG),
    )
    def k(x_hbm, nbr_hbm, out_hbm, idx_v, gat_v, *sems):
        isem = sems[:NI]
        gsem = sems[NI:NI + NG]
        ssem = sems[NI + NG:]
        wid = lax.axis_index("s") * NC + lax.axis_index("c")
        e0w = wid * NPW * K

        def issue_idx(g, si):
            eg = e_chunk + e0w + g * GK
            pltpu.async_copy(nbr_hbm.at[pl.ds(eg, GK)], idx_v.at[si], isem[si])

        def wait_idx(si):
            pltpu.make_async_copy(
                nbr_hbm.at[pl.ds(0, GK)], idx_v.at[si], isem[si]).wait()

        def issue_gather(g, sg, si):
            pltpu.async_copy(x_hbm.at[idx_v.at[si]], gat_v.at[sg], gsem[sg])

        def wait_gather(sg, si):
            pltpu.make_async_copy(
                x_hbm.at[idx_v.at[si]], gat_v.at[sg], gsem[sg]).wait()

        def issue_out(g, sg):
            eg = e0w + g * GK
            pltpu.async_copy(gat_v.at[sg], out_hbm.at[pl.ds(eg, GK)], ssem[sg])

        def wait_out(sg):
            pltpu.make_async_copy(
                gat_v.at[sg], out_hbm.at[pl.ds(0, GK)], ssem[sg]).wait()

        issue_idx(0, 0)
        issue_idx(1, 1)
        wait_idx(0)
        issue_gather(0, 0, 0)

        def quad_body(gg, car):
            for u in range(NG):
                g2 = gg * NG + u

                @pl.when(g2 + 2 < n_groups)
                def _():
                    issue_idx(g2 + 2, (u + 2) % NI)

                @pl.when(g2 + 1 < n_groups)
                def _():
                    wait_idx((u + 1) % NI)

                    @pl.when(g2 + 1 >= NG)
                    def _():
                        wait_out((u + 1) % NG)

                    issue_gather(g2 + 1, (u + 1) % NG, (u + 1) % NI)

                wait_gather(u, u % NI)
                issue_out(g2, u)
            return car

        lax.fori_loop(0, n_groups // NG, quad_body, 0)
        for t in range(min(NG, n_groups)):
            wait_out((n_groups - 1 - t) % NG)

    return k(x2, nbr1)


def _tc_body(x_ref, nbr_ref, alias_ref, out_ref):
    del alias_ref
    xb = x_ref[...]
    K, D = nbr_ref.shape[1], x_ref.shape[1]
    br = lax.broadcast_in_dim(xb, (xb.shape[0], K, D), (0, 2))
    out_ref[:, :, 0:D] = br
    out_ref[:, :, D:2 * D] = nbr_ref[...] - br


def _tc_body0(x_ref, nbr_ref, out_ref):
    _tc_body(x_ref, nbr_ref, None, out_ref)


@functools.partial(jax.jit, static_argnums=(3, 4, 5, 6, 7))
def _tc_stage(x2, nbr_c, prev_out, co, BN, D, K, CN):
    nblk = CN // _BLK
    base = co * nblk
    grid = (nblk,)
    x_spec = pl.BlockSpec((_BLK, D), lambda i, _b=base: (i + _b, 0))
    nbr_spec = pl.BlockSpec((_BLK, K, D), lambda i: (i, 0, 0))
    out_spec = pl.BlockSpec((_BLK, K, 2 * D), lambda i, _b=base: (i + _b, 0, 0))
    out_shape = jax.ShapeDtypeStruct((BN, K, 2 * D), jnp.float32)
    nbr3 = nbr_c.reshape(CN, K, D)
    if prev_out is None:
        return pl.pallas_call(
            _tc_body0, grid=grid,
            in_specs=[x_spec, nbr_spec],
            out_specs=out_spec, out_shape=out_shape,
        )(x2, nbr3)
    return pl.pallas_call(
        _tc_body, grid=grid,
        in_specs=[x_spec, nbr_spec, pl.BlockSpec(memory_space=pl.ANY)],
        out_specs=out_spec, out_shape=out_shape,
        input_output_aliases={2: 0},
    )(x2, nbr3, prev_out)


def kernel(X_inputs, nn_idx):
    B, N, D = X_inputs.shape
    K = nn_idx.shape[-1]
    BN = B * N
    CN = BN // _CHUNKS
    x2 = X_inputs.reshape(BN, D)
    offs = (jnp.arange(B, dtype=jnp.int32) * N).reshape(B, 1, 1)
    nbr1 = (nn_idx.astype(jnp.int32) + offs).reshape(BN * K)
    gathered = [_sc_gather(x2, nbr1, c, CN, D, K, 8) for c in range(_CHUNKS)]
    out = None
    for c in range(_CHUNKS):
        out = _tc_stage(x2, gathered[c], out, c, BN, D, K, CN)
    return out.reshape(B, N, K, 2 * D)
